# baseline (speedup 1.0000x reference)
import functools
import math

import jax
import jax.numpy as jnp
import numpy as np
from jax.experimental import pallas as pl
from jax.experimental.pallas import tpu as pltpu

_NEG = -1e30


def _threefry2x32(k0, k1, x0, x1):
    x0 = x0.astype(np.uint32).copy()
    x1 = x1.astype(np.uint32).copy()
    ks = [np.uint32(k0), np.uint32(k1),
          np.uint32(np.uint32(k0) ^ np.uint32(k1) ^ np.uint32(0x1BD11BDA))]
    rot = ((13, 15, 26, 6), (17, 29, 16, 24))
    x0 = (x0 + ks[0]).astype(np.uint32)
    x1 = (x1 + ks[1]).astype(np.uint32)
    for i in range(5):
        for r in rot[i % 2]:
            x0 = (x0 + x1).astype(np.uint32)
            x1 = ((x1 << np.uint32(r)) | (x1 >> np.uint32(32 - r))).astype(np.uint32)
            x1 = x1 ^ x0
        x0 = (x0 + ks[(i + 1) % 3]).astype(np.uint32)
        x1 = (x1 + ks[(i + 2) % 3] + np.uint32(i + 1)).astype(np.uint32)
    return x0, x1


@functools.lru_cache(maxsize=None)
def _sample_count_matrix(L_Q: int, L_K: int, sample_k: int):
    s1, s2 = _threefry2x32(np.uint32(0), np.uint32(42),
                           np.array([0, 0], np.uint32),
                           np.array([0, 1], np.uint32))
    n = L_Q * sample_k
    cnt64 = np.arange(n, dtype=np.uint64)
    hi = (cnt64 >> np.uint64(32)).astype(np.uint32)
    lo = (cnt64 & np.uint64(0xFFFFFFFF)).astype(np.uint32)
    bu = _threefry2x32(s1[0], s2[0], hi, lo)
    bv = _threefry2x32(s1[1], s2[1], hi, lo)
    uu = (bu[0] ^ bu[1]).reshape(L_Q, sample_k)
    vv = (bv[0] ^ bv[1]).reshape(L_Q, sample_k)
    span = np.uint32(L_K)
    mult = np.uint32((int(np.uint32(65536) % span) ** 2) % int(span))
    idx = (((uu % span) * mult + vv % span) % span).astype(np.int32)
    cntT = np.zeros((L_K, L_Q), np.int8)
    np.add.at(cntT, (idx, np.arange(L_Q)[:, None]), 1)
    return jnp.asarray(cntT)


def _select_kernel(q_ref, k_ref, cnt_ref, mtf_ref, mti_ref, m_scr,
                   *, H, L, D, u, sample_k, kc, hpb):
    f32 = jnp.float32
    p = pl.program_id(0)
    npair = H // hpb

    @pl.when(p < npair)
    def _phase_m():
        for t in range(hpb):
            lo, hi_ = t * D, (t + 1) * D
            q = q_ref[:, lo:hi_]
            k = k_ref[:, lo:hi_]
            mmax = jnp.full((1, L), _NEG, f32)
            msum = jnp.zeros((1, L), f32)
            for c in range(L // kc):
                kchunk = k[c * kc:(c + 1) * kc, :]
                st = jax.lax.dot_general(kchunk, q, (((1,), (1,)), ((), ())),
                                         preferred_element_type=f32)
                cnt = cnt_ref[c * kc:(c + 1) * kc, :].astype(f32)
                mmax = jnp.maximum(
                    mmax,
                    jnp.max(jnp.where(cnt > 0.0, st, _NEG), axis=0,
                            keepdims=True))
                msum = msum + jnp.sum(st * cnt, axis=0, keepdims=True)
            m_stat = mmax - msum * (1.0 / sample_k)
            m_scr[pl.ds(hpb * p + t, 1), :] = m_stat

    @pl.when(p == npair)
    def _phase_select():
        flatI = jax.lax.broadcasted_iota(jnp.int32, (H, L), 1)

        def body(i, m_all):
            mx = jnp.max(m_all, axis=1, keepdims=True)
            am = jnp.min(jnp.where(m_all >= mx, flatI, L), axis=1,
                         keepdims=True)
            return jnp.where(flatI == am, _NEG, m_all)

        m_fin = jax.lax.fori_loop(0, u, body, m_scr[...])
        sel = (m_fin < -1e29).astype(f32)

        rank = sel
        s = 1
        while s < L:
            rank = rank + jnp.concatenate(
                [jnp.zeros((H, s), f32), rank[:, :L - s]], axis=1)
            s *= 2

        flat = jax.lax.broadcasted_iota(jnp.int32, (u, L), 1).astype(f32)
        slot = jax.lax.broadcasted_iota(jnp.int32, (u, 1), 0).astype(f32) + 1.0
        for h in range(H):
            oh = jnp.where((rank[h:h + 1, :] == slot)
                           & (sel[h:h + 1, :] > 0.5), 1.0, 0.0)
            mtf = jnp.sum(oh * flat, axis=1, keepdims=True)
            mtf_ref[:, h:h + 1] = mtf
            mti_ref[:, h:h + 1] = mtf.astype(jnp.int32)


def _attn_kernel(q_ref, k_ref, v_ref, mti_ref, mtf_ref, o_ref, qsel_ref,
                 *, H, L, D, u, rc, hpb):
    f32 = jnp.float32
    p = pl.program_id(0)
    for t in range(hpb):
        lo, hi_ = t * D, (t + 1) * D
        k = k_ref[:, lo:hi_]
        v = v_ref[:, lo:hi_]
        h = hpb * p + t

        for i in range(u):
            qsel_ref[i:i + 1, :] = q_ref[pl.ds(mti_ref[i, h], 1), lo:hi_]
        sc = jax.lax.dot_general(qsel_ref[...], k, (((1,), (1,)), ((), ())),
                                 preferred_element_type=f32)
        sc = sc * (1.0 / math.sqrt(D))
        col = jax.lax.broadcasted_iota(jnp.int32, (u, L), 1).astype(f32)
        hsel = (jax.lax.broadcasted_iota(jnp.int32, (H, 1), 0)
                == h).astype(f32)
        thr_h = jax.lax.dot_general(mtf_ref[...], hsel,
                                    (((1,), (0,)), ((), ())),
                                    preferred_element_type=f32,
                                    precision=jax.lax.Precision.HIGHEST)
        sc = jnp.where(col > thr_h, -jnp.inf, sc)
        sc = sc - jnp.max(sc, axis=1, keepdims=True)
        e = jnp.exp(sc)
        attn = e / jnp.sum(e, axis=1, keepdims=True)
        upd = jax.lax.dot_general(attn, v, (((1,), (0,)), ((), ())),
                                  preferred_element_type=f32)

        row = jax.lax.broadcasted_iota(jnp.int32, (rc, rc), 0)
        colr = jax.lax.broadcasted_iota(jnp.int32, (rc, rc), 1)
        tri = (row >= colr).astype(f32)
        carry = jnp.zeros((1, D), f32)
        for r in range(L // rc):
            vchunk = v[r * rc:(r + 1) * rc, :]
            local = jax.lax.dot_general(tri, vchunk, (((1,), (0,)), ((), ())),
                                        preferred_element_type=f32)
            o_ref[r * rc:(r + 1) * rc, lo:hi_] = local + carry
            carry = carry + jnp.sum(vchunk, axis=0, keepdims=True)

        for i in range(u):
            o_ref[pl.ds(mti_ref[i, h], 1), lo:hi_] = upd[i:i + 1, :]


def _prob_attn_pallas(q2, k2, v2, cntT, *, H, L, D, u, sample_k, hpb):
    npair = H // hpb
    bspec = pl.BlockSpec((L, hpb * D),
                         lambda p: (0, jnp.minimum(p, npair - 1)))
    sel_kern = functools.partial(_select_kernel, H=H, L=L, D=D, u=u,
                                 sample_k=sample_k, kc=512, hpb=hpb)
    mtf, mti = pl.pallas_call(
        sel_kern,
        grid=(npair + 1,),
        in_specs=[bspec, bspec, pl.BlockSpec((L, L), lambda p: (0, 0))],
        out_specs=[pl.BlockSpec((u, H), lambda p: (0, 0)),
                   pl.BlockSpec((u, H), lambda p: (0, 0))],
        out_shape=[jax.ShapeDtypeStruct((u, H), jnp.float32),
                   jax.ShapeDtypeStruct((u, H), jnp.int32)],
        scratch_shapes=[pltpu.VMEM((H, L), jnp.float32)],
    )(q2, k2, cntT)

    attn_kern = functools.partial(_attn_kernel, H=H, L=L, D=D, u=u,
                                  rc=256, hpb=hpb)
    bspec2 = pl.BlockSpec((L, hpb * D), lambda p: (0, p))
    out = pl.pallas_call(
        attn_kern,
        grid=(npair,),
        in_specs=[bspec2, bspec2, bspec2,
                  pl.BlockSpec(memory_space=pltpu.SMEM),
                  pl.BlockSpec((u, H), lambda p: (0, 0))],
        out_specs=bspec2,
        out_shape=jax.ShapeDtypeStruct((L, H * D), jnp.float32),
        scratch_shapes=[pltpu.VMEM((u, D), jnp.float32)],
    )(q2, k2, v2, mti, mtf)
    return out


def kernel(queries, keys, values, attn_mask):
    B, L, H, D = queries.shape
    L_K = keys.shape[1]
    factor = 5
    sample_k = max(1, min(factor * int(np.ceil(np.log(L_K))), L_K))
    u = max(1, min(factor * int(np.ceil(np.log(L))), L))
    cntT = _sample_count_matrix(L, L_K, sample_k)

    q2 = jnp.reshape(queries, (L, H * D))
    k2 = jnp.reshape(keys, (L, H * D))
    v2 = jnp.reshape(values, (L, H * D))
    out = _prob_attn_pallas(q2, k2, v2, cntT, H=H, L=L, D=D, u=u,
                            sample_k=sample_k, hpb=2)
    return jnp.reshape(out, (B, L, H, D))


# speedup vs baseline: 1.1221x; 1.1221x over previous
import functools
import math

import jax
import jax.numpy as jnp
import numpy as np
from jax.experimental import pallas as pl
from jax.experimental.pallas import tpu as pltpu

_NEG = -1e30


def _threefry2x32(k0, k1, x0, x1):
    x0 = x0.astype(np.uint32).copy()
    x1 = x1.astype(np.uint32).copy()
    ks = [np.uint32(k0), np.uint32(k1),
          np.uint32(np.uint32(k0) ^ np.uint32(k1) ^ np.uint32(0x1BD11BDA))]
    rot = ((13, 15, 26, 6), (17, 29, 16, 24))
    x0 = (x0 + ks[0]).astype(np.uint32)
    x1 = (x1 + ks[1]).astype(np.uint32)
    for i in range(5):
        for r in rot[i % 2]:
            x0 = (x0 + x1).astype(np.uint32)
            x1 = ((x1 << np.uint32(r)) | (x1 >> np.uint32(32 - r))).astype(np.uint32)
            x1 = x1 ^ x0
        x0 = (x0 + ks[(i + 1) % 3]).astype(np.uint32)
        x1 = (x1 + ks[(i + 2) % 3] + np.uint32(i + 1)).astype(np.uint32)
    return x0, x1


@functools.lru_cache(maxsize=None)
def _sample_count_matrix(L_Q: int, L_K: int, sample_k: int):
    s1, s2 = _threefry2x32(np.uint32(0), np.uint32(42),
                           np.array([0, 0], np.uint32),
                           np.array([0, 1], np.uint32))
    n = L_Q * sample_k
    cnt64 = np.arange(n, dtype=np.uint64)
    hi = (cnt64 >> np.uint64(32)).astype(np.uint32)
    lo = (cnt64 & np.uint64(0xFFFFFFFF)).astype(np.uint32)
    bu = _threefry2x32(s1[0], s2[0], hi, lo)
    bv = _threefry2x32(s1[1], s2[1], hi, lo)
    uu = (bu[0] ^ bu[1]).reshape(L_Q, sample_k)
    vv = (bv[0] ^ bv[1]).reshape(L_Q, sample_k)
    span = np.uint32(L_K)
    mult = np.uint32((int(np.uint32(65536) % span) ** 2) % int(span))
    idx = (((uu % span) * mult + vv % span) % span).astype(np.int32)
    cntT = np.zeros((L_K, L_Q), np.int16)
    np.add.at(cntT, (idx, np.arange(L_Q)[:, None]), 1)
    cntf = cntT.astype(np.float32)
    nega = np.where(cntT > 0, 0.0, -2e30).astype(np.float32)
    return jnp.asarray(cntf), jnp.asarray(nega)


def _select_kernel(q_ref, k_ref, cnt_ref, nega_ref, mtf_ref, mti_ref, m_scr,
                   *, H, L, D, u, sample_k, kc, hpb):
    f32 = jnp.float32
    p = pl.program_id(0)
    npair = H // hpb

    @pl.when(p < npair)
    def _phase_m():
        for t in range(hpb):
            lo, hi_ = t * D, (t + 1) * D
            q = q_ref[:, lo:hi_]
            k = k_ref[:, lo:hi_]
            mmax = jnp.full((1, L), _NEG, f32)
            msum = jnp.zeros((1, L), f32)
            for c in range(L // kc):
                kchunk = k[c * kc:(c + 1) * kc, :]
                st = jax.lax.dot_general(kchunk, q, (((1,), (1,)), ((), ())),
                                         preferred_element_type=f32)
                mmax = jnp.maximum(
                    mmax,
                    jnp.max(st + nega_ref[c * kc:(c + 1) * kc, :], axis=0,
                            keepdims=True))
                msum = msum + jnp.sum(st * cnt_ref[c * kc:(c + 1) * kc, :],
                                      axis=0, keepdims=True)
            m_stat = mmax - msum * (1.0 / sample_k)
            m_scr[pl.ds(hpb * p + t, 1), :] = m_stat

    @pl.when(p == npair)
    def _phase_select():
        flatI = jax.lax.broadcasted_iota(jnp.int32, (H, L), 1)

        def body(i, m_all):
            mx = jnp.max(m_all, axis=1, keepdims=True)
            am = jnp.min(jnp.where(m_all >= mx, flatI, L), axis=1,
                         keepdims=True)
            return jnp.where(flatI == am, _NEG, m_all)

        m_fin = jax.lax.fori_loop(0, u, body, m_scr[...])
        sel = (m_fin < -1e29).astype(f32)

        rank = sel
        s = 1
        while s < L:
            rank = rank + jnp.concatenate(
                [jnp.zeros((H, s), f32), rank[:, :L - s]], axis=1)
            s *= 2

        flat = jax.lax.broadcasted_iota(jnp.int32, (u, L), 1).astype(f32)
        slot = jax.lax.broadcasted_iota(jnp.int32, (u, 1), 0).astype(f32) + 1.0
        for h in range(H):
            oh = jnp.where((rank[h:h + 1, :] == slot)
                           & (sel[h:h + 1, :] > 0.5), 1.0, 0.0)
            mtf = jnp.sum(oh * flat, axis=1, keepdims=True)
            mtf_ref[:, h:h + 1] = mtf
            mti_ref[:, h:h + 1] = mtf.astype(jnp.int32)


def _attn_kernel(q_ref, k_ref, v_ref, mti_ref, mtf_ref, o_ref, qsel_ref,
                 *, H, L, D, u, rc, hpb):
    f32 = jnp.float32
    p = pl.program_id(0)
    for t in range(hpb):
        lo, hi_ = t * D, (t + 1) * D
        k = k_ref[:, lo:hi_]
        v = v_ref[:, lo:hi_]
        h = hpb * p + t

        for i in range(u):
            qsel_ref[i:i + 1, :] = q_ref[pl.ds(mti_ref[i, h], 1), lo:hi_]
        sc = jax.lax.dot_general(qsel_ref[...], k, (((1,), (1,)), ((), ())),
                                 preferred_element_type=f32)
        sc = sc * (1.0 / math.sqrt(D))
        col = jax.lax.broadcasted_iota(jnp.int32, (u, L), 1).astype(f32)
        hsel = (jax.lax.broadcasted_iota(jnp.int32, (H, 1), 0)
                == h).astype(f32)
        thr_h = jax.lax.dot_general(mtf_ref[...], hsel,
                                    (((1,), (0,)), ((), ())),
                                    preferred_element_type=f32,
                                    precision=jax.lax.Precision.HIGHEST)
        sc = jnp.where(col > thr_h, -jnp.inf, sc)
        sc = sc - jnp.max(sc, axis=1, keepdims=True)
        e = jnp.exp(sc)
        attn = e / jnp.sum(e, axis=1, keepdims=True)
        upd = jax.lax.dot_general(attn, v, (((1,), (0,)), ((), ())),
                                  preferred_element_type=f32)

        row = jax.lax.broadcasted_iota(jnp.int32, (rc, rc), 0)
        colr = jax.lax.broadcasted_iota(jnp.int32, (rc, rc), 1)
        tri = (row >= colr).astype(f32)
        carry = jnp.zeros((1, D), f32)
        for r in range(L // rc):
            vchunk = v[r * rc:(r + 1) * rc, :]
            local = jax.lax.dot_general(tri, vchunk, (((1,), (0,)), ((), ())),
                                        preferred_element_type=f32)
            o_ref[r * rc:(r + 1) * rc, lo:hi_] = local + carry
            carry = carry + jnp.sum(vchunk, axis=0, keepdims=True)

        for i in range(u):
            o_ref[pl.ds(mti_ref[i, h], 1), lo:hi_] = upd[i:i + 1, :]


def _prob_attn_pallas(q2, k2, v2, cntf, nega, *, H, L, D, u, sample_k, hpb):
    npair = H // hpb
    bspec = pl.BlockSpec((L, hpb * D),
                         lambda p: (0, jnp.minimum(p, npair - 1)))
    sel_kern = functools.partial(_select_kernel, H=H, L=L, D=D, u=u,
                                 sample_k=sample_k, kc=512, hpb=hpb)
    mtf, mti = pl.pallas_call(
        sel_kern,
        grid=(npair + 1,),
        in_specs=[bspec, bspec, pl.BlockSpec((L, L), lambda p: (0, 0)),
                  pl.BlockSpec((L, L), lambda p: (0, 0))],
        out_specs=[pl.BlockSpec((u, H), lambda p: (0, 0)),
                   pl.BlockSpec((u, H), lambda p: (0, 0))],
        out_shape=[jax.ShapeDtypeStruct((u, H), jnp.float32),
                   jax.ShapeDtypeStruct((u, H), jnp.int32)],
        scratch_shapes=[pltpu.VMEM((H, L), jnp.float32)],
    )(q2, k2, cntf, nega)

    attn_kern = functools.partial(_attn_kernel, H=H, L=L, D=D, u=u,
                                  rc=256, hpb=hpb)
    bspec2 = pl.BlockSpec((L, hpb * D), lambda p: (0, p))
    out = pl.pallas_call(
        attn_kern,
        grid=(npair,),
        in_specs=[bspec2, bspec2, bspec2,
                  pl.BlockSpec(memory_space=pltpu.SMEM),
                  pl.BlockSpec((u, H), lambda p: (0, 0))],
        out_specs=bspec2,
        out_shape=jax.ShapeDtypeStruct((L, H * D), jnp.float32),
        scratch_shapes=[pltpu.VMEM((u, D), jnp.float32)],
    )(q2, k2, v2, mti, mtf)
    return out


def kernel(queries, keys, values, attn_mask):
    B, L, H, D = queries.shape
    L_K = keys.shape[1]
    factor = 5
    sample_k = max(1, min(factor * int(np.ceil(np.log(L_K))), L_K))
    u = max(1, min(factor * int(np.ceil(np.log(L))), L))
    cntf, nega = _sample_count_matrix(L, L_K, sample_k)

    q2 = jnp.reshape(queries, (L, H * D))
    k2 = jnp.reshape(keys, (L, H * D))
    v2 = jnp.reshape(values, (L, H * D))
    out = _prob_attn_pallas(q2, k2, v2, cntf, nega, H=H, L=L, D=D, u=u,
                            sample_k=sample_k, hpb=2)
    return jnp.reshape(out, (B, L, H, D))
